# baseline (speedup 1.0000x reference)
import functools

import jax
import jax.numpy as jnp
from jax import lax
from jax.experimental import pallas as pl
from jax.experimental.pallas import tpu as pltpu
from jax.experimental.pallas import tpu_sc as plsc

B = 16384
D = 32
V = 1000000
VCHUNK = 62464
VMAIN = 16 * VCHUNK
VTAIL = 640
IDXCHUNK = 128


@functools.cache
def _build():
    info = plsc.get_sparse_core_info()
    nc, ns = info.num_cores, info.num_subcores
    bt = B // ns
    rounds = D // nc

    mesh = plsc.VectorSubcoreMesh(core_axis_name="c", subcore_axis_name="s")

    @functools.partial(
        pl.kernel,
        out_type=(
            jax.ShapeDtypeStruct((D, B), jnp.float32),
            jax.ShapeDtypeStruct((D, B), jnp.float32),
        ),
        mesh=mesh,
        scratch_types=[
            pltpu.VMEM((bt,), jnp.int32),
            pltpu.VMEM((bt,), jnp.int32),
            pltpu.VMEM((bt,), jnp.float32),
            pltpu.VMEM((bt,), jnp.float32),
            pltpu.VMEM_SHARED((VMAIN + VTAIL,), jnp.float32),
            pltpu.VMEM_SHARED((VMAIN + VTAIL,), jnp.float32),
            pltpu.SemaphoreType.DMA,
            pltpu.SemaphoreType.DMA,
            pltpu.SemaphoreType.DMA,
        ],
    )
    def sc_gather(user_hbm, item_hbm, utabT_hbm, itabT_hbm,
                  utailT_hbm, itailT_hbm, uoutT_hbm, ioutT_hbm,
                  uids, iids, uvals, ivals, ushared, ishared,
                  usem, isem, gsem):
        c = lax.axis_index("c")
        s = lax.axis_index("s")
        bbase = s * bt
        pltpu.sync_copy(user_hbm.at[pl.ds(bbase, bt)], uids)
        pltpu.sync_copy(item_hbm.at[pl.ds(bbase, bt)], iids)

        vbase = s * VCHUNK

        NSUB = 4
        SUB = VCHUNK // NSUB

        def fire_stage(tabT_hbm, tailT_hbm, shared, d, sem):
            for q in range(NSUB):
                sl = pl.ds(vbase + q * SUB, SUB)
                pltpu.async_copy(tabT_hbm.at[d].at[sl], shared.at[sl], sem)

            @pl.when(s == ns - 1)
            def _():
                pltpu.async_copy(tailT_hbm.at[d],
                                 shared.at[pl.ds(VMAIN, VTAIL)], sem)

        def drain_stage(tabT_hbm, tailT_hbm, shared, d, sem):
            for q in range(NSUB):
                sl = pl.ds(vbase + q * SUB, SUB)
                pltpu.make_async_copy(tabT_hbm.at[d].at[sl],
                                      shared.at[sl], sem).wait()

            @pl.when(s == ns - 1)
            def _():
                pltpu.make_async_copy(tailT_hbm.at[d],
                                      shared.at[pl.ds(VMAIN, VTAIL)], sem).wait()

        def gather_out(shared, ids, vals, outT_hbm, d):
            copies = []
            for k in range(bt // IDXCHUNK):
                sl = pl.ds(k * IDXCHUNK, IDXCHUNK)
                copies.append(pltpu.async_copy(
                    shared.at[ids.at[sl]], vals.at[sl], gsem))
            for cp in copies:
                cp.wait()
            pltpu.sync_copy(vals, outT_hbm.at[d].at[pl.ds(bbase, bt)])

        fire_stage(utabT_hbm, utailT_hbm, ushared, c * rounds, usem)

        def round_body(r, carry):
            d = c * rounds + r
            drain_stage(utabT_hbm, utailT_hbm, ushared, d, usem)
            plsc.subcore_barrier()
            fire_stage(itabT_hbm, itailT_hbm, ishared, d, isem)
            gather_out(ushared, uids, uvals, uoutT_hbm, d)
            drain_stage(itabT_hbm, itailT_hbm, ishared, d, isem)
            plsc.subcore_barrier()

            @pl.when(r < rounds - 1)
            def _():
                fire_stage(utabT_hbm, utailT_hbm, ushared, d + 1, usem)

            gather_out(ishared, iids, ivals, ioutT_hbm, d)
            return carry

        lax.fori_loop(0, rounds, round_body, 0)

    def run(user, item, user_table, item_table):
        utail = jnp.pad(user_table[16 * VCHUNK:, :], ((0, VTAIL - (V - 16 * VCHUNK)), (0, 0)))
        itail = jnp.pad(item_table[16 * VCHUNK:, :], ((0, VTAIL - (V - 16 * VCHUNK)), (0, 0)))
        uoT, ioT = sc_gather(user, item, user_table.T, item_table.T,
                             utail.T, itail.T)
        return uoT.T, ioT.T

    return run


def kernel(user, item, user_table, item_table):
    return _build()(user, item, user_table, item_table)
